# baseline (speedup 1.0000x reference)
import functools

import jax
import jax.numpy as jnp
from jax import lax
from jax.experimental import pallas as pl
from jax.experimental.pallas import tpu as pltpu
from jax.experimental.pallas import tpu_sc as plsc

B = 16384
TOTAL = 819200
DU, DI = 64, 128
D = DU + DI
NKEY = 3
LEN_OUT = NKEY * B
OFF_OUT = LEN_OUT + 1
OFF_PAD = LEN_OUT + 16


def _dense_body(u_ref, i_ref, o_ref):
    o_ref[:, :DU] = u_ref[...]
    o_ref[:, DU:] = i_ref[...]


def _dense_concat(du, di):
    rows = 512
    return pl.pallas_call(
        _dense_body,
        grid=(B // rows,),
        in_specs=[
            pl.BlockSpec((rows, DU), lambda i: (i, 0)),
            pl.BlockSpec((rows, DI), lambda i: (i, 0)),
        ],
        out_specs=pl.BlockSpec((rows, D), lambda i: (i, 0)),
        out_shape=jax.ShapeDtypeStruct((B, D), jnp.float32),
    )(du, di)


def _sc_jagged(cval, coff, vval, voff, kval, koff):
    info = plsc.get_sparse_core_info()
    NC, NS = info.num_cores, info.num_subcores
    NW = NC * NS
    VC = TOTAL // NW
    OC = B // NW
    vdt = cval.dtype
    mesh = plsc.VectorSubcoreMesh(core_axis_name="c", subcore_axis_name="s")

    @functools.partial(
        pl.kernel,
        out_type=[
            jax.ShapeDtypeStruct((NKEY * TOTAL,), vdt),
            jax.ShapeDtypeStruct((LEN_OUT,), jnp.int32),
            jax.ShapeDtypeStruct((OFF_PAD,), jnp.int32),
        ],
        mesh=mesh,
        scratch_types=[
            pltpu.VMEM((VC,), vdt),
            pltpu.VMEM((OC + 16,), jnp.int32),
            pltpu.VMEM((OC,), jnp.int32),
            pltpu.VMEM((OC,), jnp.int32),
            pltpu.VMEM((16,), jnp.int32),
            pltpu.VMEM((16,), jnp.int32),
            pltpu.SemaphoreType.DMA,
        ],
    )
    def sc_kernel(cval_h, coff_h, vval_h, voff_h, kval_h, koff_h,
                  val_o, len_o, off_o,
                  vbuf, obuf, lbuf, sbuf, idxb, gbuf, sem):
        wid = lax.axis_index("s") * NC + lax.axis_index("c")
        vals = (cval_h, vval_h, kval_h)
        offs = (coff_h, voff_h, koff_h)

        def gather_elem(src_h, index):
            idxb[...] = jnp.full((16,), index, jnp.int32)
            pltpu.async_copy(src_h.at[idxb], gbuf, sem).wait()
            return gbuf[...]

        c0 = gather_elem(coff_h, 0)
        cB = gather_elem(coff_h, B)
        v0 = gather_elem(voff_h, 0)
        vB = gather_elem(voff_h, B)
        k0 = gather_elem(koff_h, 0)
        shifts = (-c0, cB - c0 - v0, cB - c0 + vB - v0 - k0)

        vbase = wid * VC
        obase = wid * OC
        for key in range(NKEY):
            pltpu.sync_copy(vals[key].at[pl.ds(vbase, VC)], vbuf)
            pltpu.sync_copy(vbuf, val_o.at[pl.ds(key * TOTAL + vbase, VC)])

            pltpu.sync_copy(offs[key].at[pl.ds(obase, OC)],
                            obuf.at[pl.ds(0, OC)])
            obuf[pl.ds(OC, 16)] = gather_elem(offs[key], obase + OC)
            sh = shifts[key]
            for j in range(OC // 16):
                a = obuf[pl.ds(j * 16, 16)]
                b = obuf[pl.ds(j * 16 + 1, 16)]
                lbuf[pl.ds(j * 16, 16)] = b - a
                sbuf[pl.ds(j * 16, 16)] = a + sh
            pltpu.sync_copy(lbuf, len_o.at[pl.ds(key * B + obase, OC)])
            pltpu.sync_copy(sbuf, off_o.at[pl.ds(key * B + obase, OC)])

        @pl.when(wid == NW - 1)
        def _():
            kB = gather_elem(koff_h, B)
            sbuf[pl.ds(0, 16)] = kB + shifts[2]
            pltpu.sync_copy(sbuf.at[pl.ds(0, 16)],
                            off_o.at[pl.ds(LEN_OUT, 16)])

    return sc_kernel(cval, coff, vval, voff, kval, koff)


def kernel(dense_user, dense_item, seq_clicks__values, seq_clicks__offsets,
           seq_views__values, seq_views__offsets, seq_cart__values,
           seq_cart__offsets):
    values, lengths, off_pad = _sc_jagged(
        seq_clicks__values, seq_clicks__offsets,
        seq_views__values, seq_views__offsets,
        seq_cart__values, seq_cart__offsets)
    dense = _dense_concat(dense_user, dense_item)
    return dense, values, lengths, off_pad[:OFF_OUT]


# speedup vs baseline: 1.0009x; 1.0009x over previous
import functools

import jax
import jax.numpy as jnp
from jax import lax
from jax.experimental import pallas as pl
from jax.experimental.pallas import tpu as pltpu
from jax.experimental.pallas import tpu_sc as plsc

B = 16384
TOTAL = 819200
DU, DI = 64, 128
D = DU + DI
NKEY = 3
LEN_OUT = NKEY * B
OFF_OUT = LEN_OUT + 1
OFF_PAD = LEN_OUT + 16
VCHUNK = 12800


def _dense_body(u_ref, i_ref, o_ref):
    o_ref[:, :DU] = u_ref[...]
    o_ref[:, DU:] = i_ref[...]


def _dense_concat(du, di):
    rows = 512
    return pl.pallas_call(
        _dense_body,
        grid=(B // rows,),
        in_specs=[
            pl.BlockSpec((rows, DU), lambda i: (i, 0)),
            pl.BlockSpec((rows, DI), lambda i: (i, 0)),
        ],
        out_specs=pl.BlockSpec((rows, D), lambda i: (i, 0)),
        out_shape=jax.ShapeDtypeStruct((B, D), jnp.float32),
    )(du, di)


def _sc_jagged(cval, coff, vval, voff, kval, koff):
    info = plsc.get_sparse_core_info()
    NC, NS = info.num_cores, info.num_subcores
    NW = NC * NS
    VC = TOTAL // NW
    OC = B // NW
    NCH = VC // VCHUNK
    vdt = cval.dtype
    mesh = plsc.VectorSubcoreMesh(core_axis_name="c", subcore_axis_name="s")

    scratch = (
        [pltpu.VMEM((VCHUNK,), vdt) for _ in range(2)]
        + [pltpu.VMEM((OC + 16,), jnp.int32) for _ in range(NKEY)]
        + [pltpu.VMEM((OC,), jnp.int32) for _ in range(NKEY)]
        + [pltpu.VMEM((OC,), jnp.int32) for _ in range(NKEY)]
        + [pltpu.VMEM((16,), jnp.int32) for _ in range(9)]
        + [pltpu.VMEM((16,), jnp.int32) for _ in range(10)]
        + [pltpu.SemaphoreType.DMA for _ in range(5)]
    )

    @functools.partial(
        pl.kernel,
        out_type=[
            jax.ShapeDtypeStruct((NKEY * TOTAL,), vdt),
            jax.ShapeDtypeStruct((LEN_OUT,), jnp.int32),
            jax.ShapeDtypeStruct((OFF_PAD,), jnp.int32),
        ],
        mesh=mesh,
        scratch_types=scratch,
    )
    def sc_kernel(cval_h, coff_h, vval_h, voff_h, kval_h, koff_h,
                  val_o, len_o, off_o, *rest):
        rest = list(rest)
        vbuf = [rest.pop(0) for _ in range(2)]
        obuf = [rest.pop(0) for _ in range(NKEY)]
        lbuf = [rest.pop(0) for _ in range(NKEY)]
        sbuf = [rest.pop(0) for _ in range(NKEY)]
        idxb = [rest.pop(0) for _ in range(9)]
        gbuf = [rest.pop(0) for _ in range(10)]
        sem_r0, sem_r1, sem_w0, sem_w1, sem_s = rest

        wid = lax.axis_index("s") * NC + lax.axis_index("c")
        vals = (cval_h, vval_h, kval_h)
        offs = (coff_h, voff_h, koff_h)
        vbase = wid * VC
        obase = wid * OC

        small = []

        def gather_elem_start(slot, src_h, index):
            idxb[slot][...] = jnp.full((16,), index, jnp.int32)
            small.append(
                pltpu.async_copy(src_h.at[idxb[slot]], gbuf[slot], sem_s))

        for key in range(NKEY):
            small.append(pltpu.async_copy(
                offs[key].at[pl.ds(obase, OC)],
                obuf[key].at[pl.ds(0, OC)], sem_s))
            gather_elem_start(key, offs[key], obase + OC)
        gather_elem_start(3, coff_h, 0)
        gather_elem_start(4, coff_h, B)
        gather_elem_start(5, voff_h, 0)
        gather_elem_start(6, voff_h, B)
        gather_elem_start(7, koff_h, 0)
        gather_elem_start(8, koff_h, B)

        sem_r = (sem_r0, sem_r1)
        sem_w = (sem_w0, sem_w1)
        chunks = []
        for key in range(NKEY):
            for c in range(NCH):
                src = vbase + c * VCHUNK
                chunks.append((vals[key], src, key * TOTAL + src))
        NT = len(chunks)
        reads = [None] * NT
        writes = [None] * NT

        def start_read(c):
            s = c & 1
            src_h, src, _ = chunks[c]
            reads[c] = pltpu.async_copy(
                src_h.at[pl.ds(src, VCHUNK)], vbuf[s], sem_r[s])

        start_read(0)
        for c in range(NT):
            s = c & 1
            reads[c].wait()
            if c + 1 < NT:
                if c >= 1:
                    writes[c - 1].wait()
                start_read(c + 1)
            _, _, dst = chunks[c]
            writes[c] = pltpu.async_copy(
                vbuf[s], val_o.at[pl.ds(dst, VCHUNK)], sem_w[s])
        writes[NT - 2].wait()
        writes[NT - 1].wait()

        for h in small:
            h.wait()
        c0 = gbuf[3][...]
        cB = gbuf[4][...]
        v0 = gbuf[5][...]
        vB = gbuf[6][...]
        k0 = gbuf[7][...]
        kB = gbuf[8][...]
        shifts = (-c0, cB - c0 - v0, cB - c0 + vB - v0 - k0)

        outw = []
        for key in range(NKEY):
            ob = obuf[key]
            ob[pl.ds(OC, 16)] = gbuf[key][...]
            sh = shifts[key]
            for j in range(OC // 16):
                a = ob[pl.ds(j * 16, 16)]
                b = ob[pl.ds(j * 16 + 1, 16)]
                lbuf[key][pl.ds(j * 16, 16)] = b - a
                sbuf[key][pl.ds(j * 16, 16)] = a + sh
            outw.append(pltpu.async_copy(
                lbuf[key], len_o.at[pl.ds(key * B + obase, OC)], sem_s))
            outw.append(pltpu.async_copy(
                sbuf[key], off_o.at[pl.ds(key * B + obase, OC)], sem_s))

        @pl.when(wid == NW - 1)
        def _():
            gbuf[9][...] = kB + shifts[2]
            pltpu.async_copy(gbuf[9], off_o.at[pl.ds(LEN_OUT, 16)],
                             sem_s).wait()

        for h in outw:
            h.wait()

    return sc_kernel(cval, coff, vval, voff, kval, koff)


def kernel(dense_user, dense_item, seq_clicks__values, seq_clicks__offsets,
           seq_views__values, seq_views__offsets, seq_cart__values,
           seq_cart__offsets):
    values, lengths, off_pad = _sc_jagged(
        seq_clicks__values, seq_clicks__offsets,
        seq_views__values, seq_views__offsets,
        seq_cart__values, seq_cart__offsets)
    dense = _dense_concat(dense_user, dense_item)
    return dense, values, lengths, off_pad[:OFF_OUT]


# speedup vs baseline: 1.9603x; 1.9585x over previous
import functools

import jax
import jax.numpy as jnp
from jax import lax
from jax.experimental import pallas as pl
from jax.experimental.pallas import tpu as pltpu
from jax.experimental.pallas import tpu_sc as plsc

B = 16384
TOTAL = 819200
DU, DI = 64, 128
D = DU + DI
NKEY = 3
LEN_OUT = NKEY * B
OFF_OUT = LEN_OUT + 1
VCHUNK = 12800


def _dense_body(u_ref, i_ref, o_ref):
    o_ref[0:DU, :] = u_ref[...]
    o_ref[DU:, :] = jnp.swapaxes(i_ref[...], 0, 1)


def _dense_concat(du, di):
    cols = 2048
    out_t = pl.pallas_call(
        _dense_body,
        grid=(B // cols,),
        in_specs=[
            pl.BlockSpec((DU, cols), lambda j: (0, j)),
            pl.BlockSpec((cols, DI), lambda j: (j, 0)),
        ],
        out_specs=pl.BlockSpec((D, cols), lambda j: (0, j)),
        out_shape=jax.ShapeDtypeStruct((D, B), jnp.float32),
    )(du.T, di)
    return out_t.T


def _meta_body(o1_ref, o2_ref, o3_ref, len_ref, off_ref):
    o1 = o1_ref[...]
    o2 = o2_ref[...]
    o3 = o3_ref[...]
    len_ref[pl.ds(0, B)] = o1[1:] - o1[:-1]
    len_ref[pl.ds(B, B)] = o2[1:] - o2[:-1]
    len_ref[pl.ds(2 * B, B)] = o3[1:] - o3[:-1]
    c0 = o1_ref[0:1]
    s2 = o1_ref[B:B + 1] - c0 - o2_ref[0:1]
    s3 = o1_ref[B:B + 1] - c0 + o2_ref[B:B + 1] - o2_ref[0:1] - o3_ref[0:1]
    off_ref[pl.ds(0, B + 1)] = o1 - c0
    off_ref[pl.ds(B + 1, B)] = o2[1:] + s2
    off_ref[pl.ds(2 * B + 1, B)] = o3[1:] + s3


def _kjt_meta(coff, voff, koff):
    return pl.pallas_call(
        _meta_body,
        out_shape=[
            jax.ShapeDtypeStruct((LEN_OUT,), jnp.int32),
            jax.ShapeDtypeStruct((OFF_OUT,), jnp.int32),
        ],
    )(coff, voff, koff)


def _sc_values(cval, vval, kval):
    info = plsc.get_sparse_core_info()
    NC, NS = info.num_cores, info.num_subcores
    NW = NC * NS
    VC = TOTAL // NW
    NVB = NKEY * (VC // VCHUNK)
    vdt = cval.dtype
    mesh = plsc.VectorSubcoreMesh(core_axis_name="c", subcore_axis_name="s")

    scratch = (
        [pltpu.VMEM((VCHUNK,), vdt) for _ in range(NVB)]
        + [pltpu.SemaphoreType.DMA for _ in range(2 * NVB)]
    )

    @functools.partial(
        pl.kernel,
        out_type=jax.ShapeDtypeStruct((NKEY * TOTAL,), vdt),
        mesh=mesh,
        scratch_types=scratch,
    )
    def sc_kernel(cval_h, vval_h, kval_h, val_o, *rest):
        vbuf = list(rest[0:NVB])
        sem_r = rest[NVB:2 * NVB]
        sem_w = rest[2 * NVB:3 * NVB]

        wid = lax.axis_index("s") * NC + lax.axis_index("c")
        vals = (cval_h, vval_h, kval_h)
        vbase = wid * VC

        chunks = []
        for key in range(NKEY):
            for c in range(VC // VCHUNK):
                off = vbase + c * VCHUNK
                chunks.append((vals[key], off, key * TOTAL + off))
        reads = [pltpu.async_copy(
            src.at[pl.ds(off, VCHUNK)], vbuf[i], sem_r[i])
            for i, (src, off, _) in enumerate(chunks)]
        writes = []
        for i, (_, _, dst) in enumerate(chunks):
            reads[i].wait()
            writes.append(pltpu.async_copy(
                vbuf[i], val_o.at[pl.ds(dst, VCHUNK)], sem_w[i]))
        for h in writes:
            h.wait()

    return sc_kernel(cval, vval, kval)


def kernel(dense_user, dense_item, seq_clicks__values, seq_clicks__offsets,
           seq_views__values, seq_views__offsets, seq_cart__values,
           seq_cart__offsets):
    values = _sc_values(seq_clicks__values, seq_views__values,
                        seq_cart__values)
    dense = _dense_concat(dense_user, dense_item)
    lengths, offsets = _kjt_meta(seq_clicks__offsets, seq_views__offsets,
                                 seq_cart__offsets)
    return dense, values, lengths, offsets


# speedup vs baseline: 2.0516x; 1.0465x over previous
import functools

import jax
import jax.numpy as jnp
from jax import lax
from jax.experimental import pallas as pl
from jax.experimental.pallas import tpu as pltpu
from jax.experimental.pallas import tpu_sc as plsc

B = 16384
TOTAL = 819200
DU, DI = 64, 128
D = DU + DI
NKEY = 3
LEN_OUT = NKEY * B
OFF_OUT = LEN_OUT + 1
VCHUNK = 12800


def _dense_body(u_ref, i_ref, o1_ref, o2_ref, o3_ref,
                o_ref, len_ref, off_ref):
    o_ref[0:DU, :] = u_ref[...]
    o_ref[DU:, :] = jnp.swapaxes(i_ref[...], 0, 1)

    @pl.when(pl.program_id(0) == 0)
    def _():
        o1 = o1_ref[...]
        o2 = o2_ref[...]
        o3 = o3_ref[...]
        len_ref[pl.ds(0, B)] = o1[1:] - o1[:-1]
        len_ref[pl.ds(B, B)] = o2[1:] - o2[:-1]
        len_ref[pl.ds(2 * B, B)] = o3[1:] - o3[:-1]
        c0 = o1_ref[0:1]
        s2 = o1_ref[B:B + 1] - c0 - o2_ref[0:1]
        s3 = (o1_ref[B:B + 1] - c0 + o2_ref[B:B + 1] - o2_ref[0:1]
              - o3_ref[0:1])
        off_ref[pl.ds(0, B + 1)] = o1 - c0
        off_ref[pl.ds(B + 1, B)] = o2[1:] + s2
        off_ref[pl.ds(2 * B + 1, B)] = o3[1:] + s3


def _dense_and_meta(du, di, coff, voff, koff):
    cols = 2048
    zero = lambda j: (0,)
    out_t, lengths, offsets = pl.pallas_call(
        _dense_body,
        grid=(B // cols,),
        in_specs=[
            pl.BlockSpec((DU, cols), lambda j: (0, j)),
            pl.BlockSpec((cols, DI), lambda j: (j, 0)),
            pl.BlockSpec((B + 1,), zero),
            pl.BlockSpec((B + 1,), zero),
            pl.BlockSpec((B + 1,), zero),
        ],
        out_specs=[
            pl.BlockSpec((D, cols), lambda j: (0, j)),
            pl.BlockSpec((LEN_OUT,), zero),
            pl.BlockSpec((OFF_OUT,), zero),
        ],
        out_shape=[
            jax.ShapeDtypeStruct((D, B), jnp.float32),
            jax.ShapeDtypeStruct((LEN_OUT,), jnp.int32),
            jax.ShapeDtypeStruct((OFF_OUT,), jnp.int32),
        ],
    )(du.T, di, coff, voff, koff)
    return out_t.T, lengths, offsets


def _sc_values(cval, vval, kval):
    info = plsc.get_sparse_core_info()
    NC, NS = info.num_cores, info.num_subcores
    NW = NC * NS
    VC = TOTAL // NW
    NVB = NKEY * (VC // VCHUNK)
    vdt = cval.dtype
    mesh = plsc.VectorSubcoreMesh(core_axis_name="c", subcore_axis_name="s")

    scratch = (
        [pltpu.VMEM((VCHUNK,), vdt) for _ in range(NVB)]
        + [pltpu.SemaphoreType.DMA for _ in range(2 * NVB)]
    )

    @functools.partial(
        pl.kernel,
        out_type=jax.ShapeDtypeStruct((NKEY * TOTAL,), vdt),
        mesh=mesh,
        scratch_types=scratch,
    )
    def sc_kernel(cval_h, vval_h, kval_h, val_o, *rest):
        vbuf = list(rest[0:NVB])
        sem_r = rest[NVB:2 * NVB]
        sem_w = rest[2 * NVB:3 * NVB]

        wid = lax.axis_index("s") * NC + lax.axis_index("c")
        vals = (cval_h, vval_h, kval_h)
        vbase = wid * VC

        chunks = []
        for key in range(NKEY):
            for c in range(VC // VCHUNK):
                off = vbase + c * VCHUNK
                chunks.append((vals[key], off, key * TOTAL + off))
        reads = [pltpu.async_copy(
            src.at[pl.ds(off, VCHUNK)], vbuf[i], sem_r[i])
            for i, (src, off, _) in enumerate(chunks)]
        writes = []
        for i, (_, _, dst) in enumerate(chunks):
            reads[i].wait()
            writes.append(pltpu.async_copy(
                vbuf[i], val_o.at[pl.ds(dst, VCHUNK)], sem_w[i]))
        for h in writes:
            h.wait()

    return sc_kernel(cval, vval, kval)


def kernel(dense_user, dense_item, seq_clicks__values, seq_clicks__offsets,
           seq_views__values, seq_views__offsets, seq_cart__values,
           seq_cart__offsets):
    values = _sc_values(seq_clicks__values, seq_views__values,
                        seq_cart__values)
    dense, lengths, offsets = _dense_and_meta(
        dense_user, dense_item, seq_clicks__offsets, seq_views__offsets,
        seq_cart__offsets)
    return dense, values, lengths, offsets


# speedup vs baseline: 2.0659x; 1.0070x over previous
import functools

import jax
import jax.numpy as jnp
from jax import lax
from jax.experimental import pallas as pl
from jax.experimental.pallas import tpu as pltpu
from jax.experimental.pallas import tpu_sc as plsc

B = 16384
TOTAL = 819200
DU, DI = 64, 128
D = DU + DI
NKEY = 3
LEN_OUT = NKEY * B
OFF_OUT = LEN_OUT + 1
VCHUNK = 12800


def _dense_body(u_ref, i_ref, o1_ref, o2_ref, o3_ref,
                o_ref, len_ref, off_ref):
    o_ref[0:DU, :] = u_ref[...]
    o_ref[DU:, :] = jnp.swapaxes(i_ref[...], 0, 1)

    @pl.when(pl.program_id(0) == 0)
    def _():
        o1 = o1_ref[...]
        o2 = o2_ref[...]
        o3 = o3_ref[...]
        len_ref[pl.ds(0, B)] = o1[1:] - o1[:-1]
        len_ref[pl.ds(B, B)] = o2[1:] - o2[:-1]
        len_ref[pl.ds(2 * B, B)] = o3[1:] - o3[:-1]
        c0 = o1_ref[0:1]
        s2 = o1_ref[B:B + 1] - c0 - o2_ref[0:1]
        s3 = (o1_ref[B:B + 1] - c0 + o2_ref[B:B + 1] - o2_ref[0:1]
              - o3_ref[0:1])
        off_ref[pl.ds(0, B + 1)] = o1 - c0
        off_ref[pl.ds(B + 1, B)] = o2[1:] + s2
        off_ref[pl.ds(2 * B + 1, B)] = o3[1:] + s3


def _dense_and_meta(du, di, coff, voff, koff):
    cols = 4096
    zero = lambda j: (0,)
    out_t, lengths, offsets = pl.pallas_call(
        _dense_body,
        grid=(B // cols,),
        in_specs=[
            pl.BlockSpec((DU, cols), lambda j: (0, j)),
            pl.BlockSpec((cols, DI), lambda j: (j, 0)),
            pl.BlockSpec((B + 1,), zero),
            pl.BlockSpec((B + 1,), zero),
            pl.BlockSpec((B + 1,), zero),
        ],
        out_specs=[
            pl.BlockSpec((D, cols), lambda j: (0, j)),
            pl.BlockSpec((LEN_OUT,), zero),
            pl.BlockSpec((OFF_OUT,), zero),
        ],
        out_shape=[
            jax.ShapeDtypeStruct((D, B), jnp.float32),
            jax.ShapeDtypeStruct((LEN_OUT,), jnp.int32),
            jax.ShapeDtypeStruct((OFF_OUT,), jnp.int32),
        ],
    )(du.T, di, coff, voff, koff)
    return out_t.T, lengths, offsets


def _sc_values(cval, vval, kval):
    info = plsc.get_sparse_core_info()
    NC, NS = info.num_cores, info.num_subcores
    NW = NC * NS
    VC = TOTAL // NW
    NVB = NKEY * (VC // VCHUNK)
    vdt = cval.dtype
    mesh = plsc.VectorSubcoreMesh(core_axis_name="c", subcore_axis_name="s")

    scratch = (
        [pltpu.VMEM((VCHUNK,), vdt) for _ in range(NVB)]
        + [pltpu.SemaphoreType.DMA for _ in range(2 * NVB)]
    )

    @functools.partial(
        pl.kernel,
        out_type=jax.ShapeDtypeStruct((NKEY * TOTAL,), vdt),
        mesh=mesh,
        scratch_types=scratch,
    )
    def sc_kernel(cval_h, vval_h, kval_h, val_o, *rest):
        vbuf = list(rest[0:NVB])
        sem_r = rest[NVB:2 * NVB]
        sem_w = rest[2 * NVB:3 * NVB]

        wid = lax.axis_index("s") * NC + lax.axis_index("c")
        vals = (cval_h, vval_h, kval_h)
        vbase = wid * VC

        chunks = []
        for key in range(NKEY):
            for c in range(VC // VCHUNK):
                off = vbase + c * VCHUNK
                chunks.append((vals[key], off, key * TOTAL + off))
        reads = [pltpu.async_copy(
            src.at[pl.ds(off, VCHUNK)], vbuf[i], sem_r[i])
            for i, (src, off, _) in enumerate(chunks)]
        writes = []
        for i, (_, _, dst) in enumerate(chunks):
            reads[i].wait()
            writes.append(pltpu.async_copy(
                vbuf[i], val_o.at[pl.ds(dst, VCHUNK)], sem_w[i]))
        for h in writes:
            h.wait()

    return sc_kernel(cval, vval, kval)


def kernel(dense_user, dense_item, seq_clicks__values, seq_clicks__offsets,
           seq_views__values, seq_views__offsets, seq_cart__values,
           seq_cart__offsets):
    values = _sc_values(seq_clicks__values, seq_views__values,
                        seq_cart__values)
    dense, lengths, offsets = _dense_and_meta(
        dense_user, dense_item, seq_clicks__offsets, seq_views__offsets,
        seq_cart__offsets)
    return dense, values, lengths, offsets


# speedup vs baseline: 2.1272x; 1.0297x over previous
import functools

import jax
import jax.numpy as jnp
from jax import lax
from jax.experimental import pallas as pl
from jax.experimental.pallas import tpu as pltpu
from jax.experimental.pallas import tpu_sc as plsc

B = 16384
TOTAL = 819200
DU, DI = 64, 128
D = DU + DI
NKEY = 3
LEN_OUT = NKEY * B
OFF_OUT = LEN_OUT + 1
VCHUNK = 12800


def _dense_body(u_ref, i_ref, o1_ref, o2_ref, o3_ref,
                o_ref, len_ref, off_ref):
    o_ref[0:DU, :] = u_ref[...]
    o_ref[DU:, :] = jnp.swapaxes(i_ref[...], 0, 1)

    @pl.when(pl.program_id(0) == 0)
    def _():
        o1 = o1_ref[...]
        o2 = o2_ref[...]
        o3 = o3_ref[...]
        len_ref[pl.ds(0, B)] = o1[1:] - o1[:-1]
        len_ref[pl.ds(B, B)] = o2[1:] - o2[:-1]
        len_ref[pl.ds(2 * B, B)] = o3[1:] - o3[:-1]
        c0 = o1_ref[0:1]
        s2 = o1_ref[B:B + 1] - c0 - o2_ref[0:1]
        s3 = (o1_ref[B:B + 1] - c0 + o2_ref[B:B + 1] - o2_ref[0:1]
              - o3_ref[0:1])
        off_ref[pl.ds(0, B + 1)] = o1 - c0
        off_ref[pl.ds(B + 1, B)] = o2[1:] + s2
        off_ref[pl.ds(2 * B + 1, B)] = o3[1:] + s3


def _dense_and_meta(du, di, coff, voff, koff):
    cols = 8192
    zero = lambda j: (0,)
    out_t, lengths, offsets = pl.pallas_call(
        _dense_body,
        grid=(B // cols,),
        in_specs=[
            pl.BlockSpec((DU, cols), lambda j: (0, j)),
            pl.BlockSpec((cols, DI), lambda j: (j, 0)),
            pl.BlockSpec((B + 1,), zero),
            pl.BlockSpec((B + 1,), zero),
            pl.BlockSpec((B + 1,), zero),
        ],
        out_specs=[
            pl.BlockSpec((D, cols), lambda j: (0, j)),
            pl.BlockSpec((LEN_OUT,), zero),
            pl.BlockSpec((OFF_OUT,), zero),
        ],
        out_shape=[
            jax.ShapeDtypeStruct((D, B), jnp.float32),
            jax.ShapeDtypeStruct((LEN_OUT,), jnp.int32),
            jax.ShapeDtypeStruct((OFF_OUT,), jnp.int32),
        ],
    )(du.T, di, coff, voff, koff)
    return out_t.T, lengths, offsets


def _sc_values(cval, vval, kval):
    info = plsc.get_sparse_core_info()
    NC, NS = info.num_cores, info.num_subcores
    NW = NC * NS
    VC = TOTAL // NW
    NVB = NKEY * (VC // VCHUNK)
    vdt = cval.dtype
    mesh = plsc.VectorSubcoreMesh(core_axis_name="c", subcore_axis_name="s")

    scratch = (
        [pltpu.VMEM((VCHUNK,), vdt) for _ in range(NVB)]
        + [pltpu.SemaphoreType.DMA for _ in range(2 * NVB)]
    )

    @functools.partial(
        pl.kernel,
        out_type=jax.ShapeDtypeStruct((NKEY * TOTAL,), vdt),
        mesh=mesh,
        scratch_types=scratch,
    )
    def sc_kernel(cval_h, vval_h, kval_h, val_o, *rest):
        vbuf = list(rest[0:NVB])
        sem_r = rest[NVB:2 * NVB]
        sem_w = rest[2 * NVB:3 * NVB]

        wid = lax.axis_index("s") * NC + lax.axis_index("c")
        vals = (cval_h, vval_h, kval_h)
        vbase = wid * VC

        chunks = []
        for key in range(NKEY):
            for c in range(VC // VCHUNK):
                off = vbase + c * VCHUNK
                chunks.append((vals[key], off, key * TOTAL + off))
        reads = [pltpu.async_copy(
            src.at[pl.ds(off, VCHUNK)], vbuf[i], sem_r[i])
            for i, (src, off, _) in enumerate(chunks)]
        writes = []
        for i, (_, _, dst) in enumerate(chunks):
            reads[i].wait()
            writes.append(pltpu.async_copy(
                vbuf[i], val_o.at[pl.ds(dst, VCHUNK)], sem_w[i]))
        for h in writes:
            h.wait()

    return sc_kernel(cval, vval, kval)


def kernel(dense_user, dense_item, seq_clicks__values, seq_clicks__offsets,
           seq_views__values, seq_views__offsets, seq_cart__values,
           seq_cart__offsets):
    values = _sc_values(seq_clicks__values, seq_views__values,
                        seq_cart__values)
    dense, lengths, offsets = _dense_and_meta(
        dense_user, dense_item, seq_clicks__offsets, seq_views__offsets,
        seq_cart__offsets)
    return dense, values, lengths, offsets
